# baseline (speedup 1.0000x reference)
import jax
import jax.numpy as jnp
from jax import lax
from jax.experimental import pallas as pl
from jax.experimental.pallas import tpu as pltpu
from jax.experimental.pallas import tpu_sc as plsc

B = 4096
L = 50
NF = 26
D = 64
NC = 2
NS = 16
NW = NC * NS
RPW = B // NW
P = 8
NPH = RPW // P
SROWS = P * L
CROWS = P * NF


def _make_kernel():
    mesh = plsc.VectorSubcoreMesh(core_axis_name="c", subcore_axis_name="s")

    def body(seq_ids_hbm, ctx_ids_hbm, item_hbm, ctx_table_hbm, out_hbm,
             seq_idx_v, ctx_idx_v, rows_v, acc_v, sem):
        cid = lax.axis_index("c")
        sid = lax.axis_index("s")
        wid = sid * NC + cid
        base = wid * RPW

        pltpu.sync_copy(seq_ids_hbm.at[wid], seq_idx_v)
        pltpu.sync_copy(ctx_ids_hbm.at[wid], ctx_idx_v)

        inv_l = jnp.full((16,), 1.0 / L, jnp.float32)

        streams = [
            (item_hbm, seq_idx_v, L, SROWS, 0, inv_l),
            (ctx_table_hbm, ctx_idx_v, NF, CROWS, RPW, None),
        ]

        def run_stream(st):
            tbl, idx_v, spr, rpp, aoff, scale = streams[st]

            def gather(p, b):
                pltpu.async_copy(tbl.at[idx_v.at[p]],
                                 rows_v.at[b, pl.ds(0, rpp)], sem.at[b])

            def wait_gather(p, b):
                pltpu.make_async_copy(tbl.at[idx_v.at[p]],
                                      rows_v.at[b, pl.ds(0, rpp)],
                                      sem.at[b]).wait()

            gather(0, 0)
            gather(1, 1)

            @pl.loop(0, NPH, step=2)
            def _(p):
                for b in range(2):
                    pe = p + b
                    wait_gather(pe, b)

                    @pl.loop(0, P)
                    def _(s):
                        r0 = s * spr
                        for j in range(D // 16):
                            sl = pl.ds(j * 16, 16)
                            acc = rows_v[b, r0, sl]
                            for r in range(1, spr):
                                acc = acc + rows_v[b, r0 + r, sl]
                            if scale is not None:
                                acc = acc * scale
                            acc_v[aoff + pe * P + s, sl] = acc

                    @pl.when(pe + 2 < NPH)
                    def _():
                        gather(pe + 2, b)

        run_stream(0)
        run_stream(1)

        pltpu.sync_copy(acc_v.at[pl.ds(0, RPW)], out_hbm.at[0, pl.ds(base, RPW)])
        pltpu.sync_copy(acc_v.at[pl.ds(RPW, RPW)], out_hbm.at[1, pl.ds(base, RPW)])

    return pl.kernel(
        body,
        out_type=jax.ShapeDtypeStruct((2, B, D), jnp.float32),
        mesh=mesh,
        compiler_params=pltpu.CompilerParams(use_tc_tiling_on_sc=False),
        scratch_types=[
            pltpu.VMEM((NPH, SROWS), jnp.int32),
            pltpu.VMEM((NPH, CROWS), jnp.int32),
            pltpu.VMEM((2, SROWS, D), jnp.float32),
            pltpu.VMEM((2 * RPW, D), jnp.float32),
            pltpu.SemaphoreType.DMA((2,)),
        ],
    )


_sc_kernel = _make_kernel()


def kernel(seq_item_ids, context_ids, item_table, context_table):
    seq_ids = seq_item_ids.reshape(NW, NPH, SROWS)
    ctx_ids = context_ids.reshape(NW, NPH, CROWS)
    out = _sc_kernel(seq_ids, ctx_ids, item_table, context_table)
    return jnp.concatenate([out[0], out[1]], axis=-1)
